# baseline (speedup 1.0000x reference)
import math

import jax
import jax.numpy as jnp
from jax.experimental import pallas as pl
from jax.experimental.pallas import tpu as pltpu

_W = 128
_BQ = 512
_THETA_LOG2 = 16.0
_TWO_PI = 2.0 * math.pi


def _rope_cos_sin(pos, hd):
    d = jax.lax.broadcasted_iota(jnp.int32, (1, hd), 1)
    q = ((d // 2) * 2).astype(jnp.float32)
    freqs = jnp.exp2(-(_THETA_LOG2 / hd) * q) / _TWO_PI
    phases = pos * freqs
    ph = (phases - jnp.floor(phases)) * _TWO_PI
    return jnp.cos(ph), jnp.sin(ph)


def _rot_matrix(hd):
    r = jax.lax.broadcasted_iota(jnp.int32, (hd, hd), 0)
    c = jax.lax.broadcasted_iota(jnp.int32, (hd, hd), 1)
    c_even = (c % 2) == 0
    m = jnp.where((r == c + 1) & c_even, -1.0, 0.0)
    m = jnp.where((r == c - 1) & ~c_even, 1.0, m)
    return m.astype(jnp.float32)


def _attn_kernel(q_ref, v_ref, o_ref, kr_ref, vp_ref):
    i = pl.program_id(1)
    hd = q_ref.shape[-1]
    nsub = _BQ // _W

    @pl.when(i == 0)
    def _():
        kr_ref[...] = jnp.zeros_like(kr_ref)
        vp_ref[...] = jnp.zeros_like(vp_ref)

    qblk = q_ref[0]
    pos = i * _BQ + jax.lax.broadcasted_iota(jnp.int32, (_BQ, 1), 0)
    cos, sin = _rope_cos_sin(pos.astype(jnp.float32), hd)
    qrot = jnp.dot(qblk, _rot_matrix(hd), preferred_element_type=jnp.float32)
    qr = qblk * cos + qrot * sin

    iq = jax.lax.broadcasted_iota(jnp.int32, (_W, _W), 0)
    ik = jax.lax.broadcasted_iota(jnp.int32, (_W, _W), 1)
    mask_cur = ik < iq
    mask_prev = ik >= iq

    for j in range(nsub):
        qj = qr[j * _W:(j + 1) * _W]
        vc = v_ref[0, j * _W:(j + 1) * _W]
        if j == 0:
            kp = kr_ref[...]
            vp = vp_ref[...]
        else:
            kp = qr[(j - 1) * _W:j * _W]
            vp = v_ref[0, (j - 1) * _W:j * _W]
        s_c = jnp.dot(qj, qj.T, preferred_element_type=jnp.float32)
        s_p = jnp.dot(qj, kp.T, preferred_element_type=jnp.float32)
        s_c = jnp.where(mask_cur, s_c, 0.0)
        s_p = jnp.where(mask_prev, s_p, 0.0)
        o_ref[0, j * _W:(j + 1) * _W] = (
            jnp.dot(s_c, vc, preferred_element_type=jnp.float32)
            + jnp.dot(s_p, vp, preferred_element_type=jnp.float32)
        )

    kr_ref[...] = qr[_BQ - _W:]
    vp_ref[...] = v_ref[0, _BQ - _W:]


def kernel(Q, K, V):
    del K
    b, nh, t, hd = Q.shape
    bh = b * nh
    q = Q.reshape(bh, t, hd)
    v = V.reshape(bh, t, hd)
    nblk = t // _BQ
    out = pl.pallas_call(
        _attn_kernel,
        grid=(bh, nblk),
        in_specs=[
            pl.BlockSpec((1, _BQ, hd), lambda b_, i_: (b_, i_, 0)),
            pl.BlockSpec((1, _BQ, hd), lambda b_, i_: (b_, i_, 0)),
        ],
        out_specs=pl.BlockSpec((1, _BQ, hd), lambda b_, i_: (b_, i_, 0)),
        out_shape=jax.ShapeDtypeStruct((bh, t, hd), jnp.float32),
        scratch_shapes=[
            pltpu.VMEM((_W, hd), jnp.float32),
            pltpu.VMEM((_W, hd), jnp.float32),
        ],
        compiler_params=pltpu.CompilerParams(
            dimension_semantics=("arbitrary", "arbitrary"),
        ),
    )(q, v)
    return out.reshape(b, nh, t, hd)


# speedup vs baseline: 1.0357x; 1.0357x over previous
import math

import jax
import jax.numpy as jnp
from jax.experimental import pallas as pl
from jax.experimental.pallas import tpu as pltpu

_W = 128
_BQ = 512
_THETA_LOG2 = 16.0
_TWO_PI = 2.0 * math.pi


def _rope_cos_sin(pos, hd):
    d = jax.lax.broadcasted_iota(jnp.int32, (1, hd), 1)
    q = ((d // 2) * 2).astype(jnp.float32)
    freqs = jnp.exp2(-(_THETA_LOG2 / hd) * q) / _TWO_PI
    phases = pos * freqs
    ph = (phases - jnp.floor(phases)) * _TWO_PI
    return jnp.cos(ph), jnp.sin(ph)


def _rot_matrix(hd):
    r = jax.lax.broadcasted_iota(jnp.int32, (hd, hd), 0)
    c = jax.lax.broadcasted_iota(jnp.int32, (hd, hd), 1)
    c_even = (c % 2) == 0
    m = jnp.where((r == c + 1) & c_even, -1.0, 0.0)
    m = jnp.where((r == c - 1) & ~c_even, 1.0, m)
    return m.astype(jnp.float32)


def _attn_kernel(q_ref, v_ref, o_ref, kf_ref, vf_ref):
    i = pl.program_id(1)
    hd = q_ref.shape[-1]

    @pl.when(i == 0)
    def _():
        kf_ref[...] = jnp.zeros_like(kf_ref)
        vf_ref[...] = jnp.zeros_like(vf_ref)

    @pl.when(i > 0)
    def _():
        kf_ref[0:_W] = kf_ref[_BQ:_BQ + _W]
        vf_ref[0:_W] = vf_ref[_BQ:_BQ + _W]

    qblk = q_ref[0]
    pos = i * _BQ + jax.lax.broadcasted_iota(jnp.int32, (_BQ, 1), 0)
    cos, sin = _rope_cos_sin(pos.astype(jnp.float32), hd)
    qrot = jnp.dot(qblk, _rot_matrix(hd), preferred_element_type=jnp.float32)
    qr = qblk * cos + qrot * sin
    qr_bf = qr.astype(jnp.bfloat16)

    kf_ref[_W:] = qr_bf
    vf_ref[_W:] = v_ref[0].astype(jnp.bfloat16)

    iq = jax.lax.broadcasted_iota(jnp.int32, (_BQ, _BQ + _W), 0)
    jk = jax.lax.broadcasted_iota(jnp.int32, (_BQ, _BQ + _W), 1)
    mask = (jk >= iq) & (jk < iq + _W)

    scores = jax.lax.dot_general(
        qr_bf, kf_ref[...],
        (((1,), (1,)), ((), ())),
        preferred_element_type=jnp.float32,
    )
    scores = jnp.where(mask, scores, 0.0).astype(jnp.bfloat16)
    o_ref[0] = jax.lax.dot_general(
        scores, vf_ref[...],
        (((1,), (0,)), ((), ())),
        preferred_element_type=jnp.float32,
    )


def kernel(Q, K, V):
    del K
    b, nh, t, hd = Q.shape
    bh = b * nh
    q = Q.reshape(bh, t, hd)
    v = V.reshape(bh, t, hd)
    nblk = t // _BQ
    out = pl.pallas_call(
        _attn_kernel,
        grid=(bh, nblk),
        in_specs=[
            pl.BlockSpec((1, _BQ, hd), lambda b_, i_: (b_, i_, 0)),
            pl.BlockSpec((1, _BQ, hd), lambda b_, i_: (b_, i_, 0)),
        ],
        out_specs=pl.BlockSpec((1, _BQ, hd), lambda b_, i_: (b_, i_, 0)),
        out_shape=jax.ShapeDtypeStruct((bh, t, hd), jnp.float32),
        scratch_shapes=[
            pltpu.VMEM((_BQ + _W, hd), jnp.bfloat16),
            pltpu.VMEM((_BQ + _W, hd), jnp.bfloat16),
        ],
        compiler_params=pltpu.CompilerParams(
            dimension_semantics=("arbitrary", "arbitrary"),
        ),
    )(q, v)
    return out.reshape(b, nh, t, hd)


# speedup vs baseline: 1.4160x; 1.3672x over previous
import math

import jax
import jax.numpy as jnp
from jax.experimental import pallas as pl
from jax.experimental.pallas import tpu as pltpu

_W = 128
_BQ = 512
_THETA_LOG2 = 16.0
_TWO_PI = 2.0 * math.pi


def _rope_cos_sin(pos, hd):
    d = jax.lax.broadcasted_iota(jnp.int32, (1, hd), 1)
    q = ((d // 2) * 2).astype(jnp.float32)
    freqs = jnp.exp2(-(_THETA_LOG2 / hd) * q) / _TWO_PI
    phases = pos * freqs
    ph = (phases - jnp.floor(phases)) * _TWO_PI
    return jnp.cos(ph), jnp.sin(ph)


def _rot_matrix(hd):
    r = jax.lax.broadcasted_iota(jnp.int32, (hd, hd), 0)
    c = jax.lax.broadcasted_iota(jnp.int32, (hd, hd), 1)
    c_even = (c % 2) == 0
    m = jnp.where((r == c + 1) & c_even, -1.0, 0.0)
    m = jnp.where((r == c - 1) & ~c_even, 1.0, m)
    return m.astype(jnp.float32)


def _attn_kernel(q_ref, v_ref, o_ref,
                 cos_ref, sin_ref, mc_ref, mh_ref, kh_ref, vh_ref):
    i = pl.program_id(0)
    b = pl.program_id(1)
    hd = q_ref.shape[-1]

    @pl.when((i == 0) & (b == 0))
    def _():
        iq = jax.lax.broadcasted_iota(jnp.int32, (_BQ, _BQ), 0)
        jc = jax.lax.broadcasted_iota(jnp.int32, (_BQ, _BQ), 1)
        mc_ref[...] = ((jc < iq) & (jc >= iq - _W)).astype(jnp.bfloat16)
        iqh = jax.lax.broadcasted_iota(jnp.int32, (_BQ, _W), 0)
        jh = jax.lax.broadcasted_iota(jnp.int32, (_BQ, _W), 1)
        mh_ref[...] = (jh >= iqh).astype(jnp.bfloat16)

    @pl.when(b == 0)
    def _():
        pos = i * _BQ + jax.lax.broadcasted_iota(jnp.int32, (_BQ, 1), 0)
        cos, sin = _rope_cos_sin(pos.astype(jnp.float32), hd)
        cos_ref[...] = cos
        sin_ref[...] = sin

    @pl.when(i == 0)
    def _():
        kh_ref[b] = jnp.zeros_like(kh_ref[b])
        vh_ref[b] = jnp.zeros_like(vh_ref[b])

    khist = kh_ref[b]
    vhist = vh_ref[b]

    qblk = q_ref[0]
    qrot = jnp.dot(qblk, _rot_matrix(hd), preferred_element_type=jnp.float32)
    qr = qblk * cos_ref[...] + qrot * sin_ref[...]
    qr_bf = qr.astype(jnp.bfloat16)
    v_bf = v_ref[0].astype(jnp.bfloat16)

    kh_ref[b] = qr_bf[_BQ - _W:]
    vh_ref[b] = v_bf[_BQ - _W:]

    s_cur = jax.lax.dot_general(
        qr_bf, qr_bf, (((1,), (1,)), ((), ())),
        preferred_element_type=jnp.float32,
    )
    s_hal = jax.lax.dot_general(
        qr_bf, khist, (((1,), (1,)), ((), ())),
        preferred_element_type=jnp.float32,
    )
    s_cur = s_cur.astype(jnp.bfloat16) * mc_ref[...]
    s_hal = s_hal.astype(jnp.bfloat16) * mh_ref[...]
    o_ref[0] = (
        jax.lax.dot_general(
            s_cur, v_bf, (((1,), (0,)), ((), ())),
            preferred_element_type=jnp.float32,
        )
        + jax.lax.dot_general(
            s_hal, vhist, (((1,), (0,)), ((), ())),
            preferred_element_type=jnp.float32,
        )
    )


def kernel(Q, K, V):
    del K
    b, nh, t, hd = Q.shape
    bh = b * nh
    q = Q.reshape(bh, t, hd)
    v = V.reshape(bh, t, hd)
    nblk = t // _BQ
    out = pl.pallas_call(
        _attn_kernel,
        grid=(nblk, bh),
        in_specs=[
            pl.BlockSpec((1, _BQ, hd), lambda i_, b_: (b_, i_, 0)),
            pl.BlockSpec((1, _BQ, hd), lambda i_, b_: (b_, i_, 0)),
        ],
        out_specs=pl.BlockSpec((1, _BQ, hd), lambda i_, b_: (b_, i_, 0)),
        out_shape=jax.ShapeDtypeStruct((bh, t, hd), jnp.float32),
        scratch_shapes=[
            pltpu.VMEM((_BQ, hd), jnp.float32),
            pltpu.VMEM((_BQ, hd), jnp.float32),
            pltpu.VMEM((_BQ, _BQ), jnp.bfloat16),
            pltpu.VMEM((_BQ, _W), jnp.bfloat16),
            pltpu.VMEM((bh, _W, hd), jnp.bfloat16),
            pltpu.VMEM((bh, _W, hd), jnp.bfloat16),
        ],
        compiler_params=pltpu.CompilerParams(
            dimension_semantics=("arbitrary", "arbitrary"),
        ),
    )(q, v)
    return out.reshape(b, nh, t, hd)
